# baseline (speedup 1.0000x reference)
import functools

import jax
import jax.numpy as jnp
from jax import lax
from jax.experimental import pallas as pl
from jax.experimental.pallas import tpu as pltpu
from jax.experimental.pallas import tpu_sc as plsc

NC = 2
NS = 16
LANES = 16
NW = NC * NS
CH = 128


def _sc_mesh():
    return plsc.VectorSubcoreMesh(
        core_axis_name="c", subcore_axis_name="s",
        num_cores=NC, num_subcores=NS)


NB = 4


def _gather_rows(table, idx):
    n_rows, d = table.shape
    dt = table.dtype
    e = idx.shape[0]
    ew = e // NW
    n_full = ew // CH
    tail = ew - n_full * CH
    n_grp = n_full // NB
    rest = n_full - n_grp * NB

    @functools.partial(
        pl.kernel,
        out_type=jax.ShapeDtypeStruct((e, d), dt),
        mesh=_sc_mesh(),
        scratch_types=[
            pltpu.VMEM((ew,), jnp.int32),
            *[pltpu.VMEM((CH, d), dt) for _ in range(NB)],
            pltpu.VMEM((max(tail, 8), d), dt),
            pltpu.SemaphoreType.DMA,
            pltpu.SemaphoreType.DMA,
        ],
        compiler_params=pltpu.CompilerParams(use_tc_tiling_on_sc=False),
    )
    def k(table_hbm, idx_hbm, out_hbm, idx_all, *bufs):
        *rows, tail_v, gsem, wsem = bufs
        wid = lax.axis_index("s") * NC + lax.axis_index("c")
        base = wid * ew
        pltpu.sync_copy(idx_hbm.at[pl.ds(base, ew)], idx_all)

        def run_group(cb, nb):
            gds = [
                pltpu.async_copy(
                    table_hbm.at[idx_all.at[pl.ds(cb + b * CH, CH)]],
                    rows[b], gsem)
                for b in range(nb)
            ]
            wds = []
            for b in range(nb):
                gds[b].wait()
                wds.append(pltpu.async_copy(
                    rows[b], out_hbm.at[pl.ds(base + cb + b * CH, CH)],
                    wsem))
            for wd in wds:
                wd.wait()

        def body(g, carry):
            run_group(g * (NB * CH), NB)
            return carry

        lax.fori_loop(0, n_grp, body, 0)
        if rest:
            run_group(n_grp * (NB * CH), rest)
        if tail:
            off = n_full * CH
            pltpu.async_copy(
                table_hbm.at[idx_all.at[pl.ds(off, tail)]],
                tail_v.at[pl.ds(0, tail)], gsem).wait()
            pltpu.async_copy(
                tail_v.at[pl.ds(0, tail)],
                out_hbm.at[pl.ds(base + off, tail)], wsem).wait()

    return k(table, idx)


def _scatter_add_parts(vals, dst, n_nodes):
    e, h = vals.shape
    n_chunks = e // CH
    per_w = -(-n_chunks // NW)
    rpt = -(-n_nodes // (8 * NS)) * 8
    npad = rpt * NS

    ew = e // NW
    n_full = ew // CH
    tail = ew - n_full * CH
    n_grp = n_full // NB
    rest = n_full - n_grp * NB

    @functools.partial(
        pl.kernel,
        out_type=jax.ShapeDtypeStruct((NC * npad, h), jnp.float32),
        mesh=_sc_mesh(),
        scratch_types=[
            *[pltpu.VMEM((CH,), jnp.int32) for _ in range(NB)],
            *[pltpu.VMEM((CH, h), jnp.float32) for _ in range(NB)],
            pltpu.VMEM((max(tail, 8),), jnp.int32),
            pltpu.VMEM((max(tail, 8), h), jnp.float32),
            pltpu.VMEM((rpt, h), jnp.float32),
            pltpu.VMEM_SHARED((npad, h), jnp.float32),
            pltpu.SemaphoreType.DMA,
            pltpu.SemaphoreType.DMA,
        ],
        compiler_params=pltpu.CompilerParams(use_tc_tiling_on_sc=False),
    )
    def k(vals_hbm, dst_hbm, out_hbm, *bufs):
        idxs = bufs[:NB]
        rows = bufs[NB:2 * NB]
        tidx, trow, acc_v, acc_sh, lsem, ssem = bufs[2 * NB:]
        cid = lax.axis_index("c")
        sid = lax.axis_index("s")
        wid = sid * NC + cid
        base = wid * ew

        zv = jnp.zeros((LANES,), jnp.float32)

        def zbody(r, carry):
            for j in range(h // LANES):
                acc_v[r, pl.ds(j * LANES, LANES)] = zv
            return carry

        lax.fori_loop(0, rpt, zbody, 0)
        pltpu.sync_copy(acc_v, acc_sh.at[pl.ds(sid * rpt, rpt)])
        plsc.subcore_barrier()

        def run_group(cb, nb):
            lds = []
            for b in range(nb):
                off = base + cb + b * CH
                lds.append((
                    pltpu.async_copy(dst_hbm.at[pl.ds(off, CH)],
                                     idxs[b], lsem),
                    pltpu.async_copy(vals_hbm.at[pl.ds(off, CH)],
                                     rows[b], lsem)))
            sds = []
            for b in range(nb):
                lds[b][0].wait()
                lds[b][1].wait()
                sds.append(pltpu.async_copy(
                    rows[b], acc_sh.at[idxs[b]], ssem, add=True))
            for sd in sds:
                sd.wait()

        def body(g, carry):
            run_group(g * (NB * CH), NB)
            return carry

        lax.fori_loop(0, n_grp, body, 0)
        if rest:
            run_group(n_grp * (NB * CH), rest)
        if tail:
            off = base + n_full * CH
            pltpu.async_copy(dst_hbm.at[pl.ds(off, tail)], tidx, lsem).wait()
            pltpu.async_copy(vals_hbm.at[pl.ds(off, tail)], trow, lsem).wait()
            pltpu.async_copy(trow, acc_sh.at[tidx], ssem, add=True).wait()
        plsc.subcore_barrier()

        pltpu.sync_copy(acc_sh.at[pl.ds(sid * rpt, rpt)], acc_v)
        pltpu.sync_copy(
            acc_v, out_hbm.at[pl.ds(cid * npad + sid * rpt, rpt)])

    return k(vals, dst).reshape(NC, npad, h)[:, :n_nodes, :]


def _edge_messages(hs, ea_ext, w2d, block=2000):
    e, d = hs.shape
    s_ext = ea_ext.shape[1]
    h = w2d.shape[1] // s_ext
    grid = e // block
    r_bcast = jnp.kron(jnp.eye(s_ext, dtype=jnp.float32),
                       jnp.ones((1, h), jnp.float32))
    r_sum = jnp.kron(jnp.ones((s_ext, 1), jnp.float32),
                     jnp.eye(h, dtype=jnp.float32))

    def body(hs_ref, ea_ref, w_ref, rb_ref, rs_ref, o_ref):
        t = jnp.dot(hs_ref[...], w_ref[...],
                    preferred_element_type=jnp.float32)
        eab = jnp.dot(ea_ref[...], rb_ref[...],
                      preferred_element_type=jnp.float32)
        o_ref[...] = jnp.dot(t * eab, rs_ref[...],
                             preferred_element_type=jnp.float32)

    return pl.pallas_call(
        body,
        grid=(grid,),
        in_specs=[
            pl.BlockSpec((block, d), lambda i: (i, 0)),
            pl.BlockSpec((block, s_ext), lambda i: (i, 0)),
            pl.BlockSpec(w2d.shape, lambda i: (0, 0)),
            pl.BlockSpec(r_bcast.shape, lambda i: (0, 0)),
            pl.BlockSpec(r_sum.shape, lambda i: (0, 0)),
        ],
        out_specs=pl.BlockSpec((block, h), lambda i: (i, 0)),
        out_shape=jax.ShapeDtypeStruct((e, h), jnp.float32),
    )(hs, ea_ext, w2d, r_bcast, r_sum)


def _node_update(agg_parts, x, root, b, out_width, block=2000):
    n, d = x.shape
    h = root.shape[1]
    grid = n // block

    def body(a_ref, x_ref, r_ref, b_ref, o_ref):
        agg = a_ref[0] + a_ref[1]
        xr = jnp.dot(x_ref[...], r_ref[...],
                     preferred_element_type=jnp.float32)
        hv = jnp.maximum(agg + xr + b_ref[...], 0.0)
        o_ref[...] = jnp.concatenate(
            [hv, jnp.zeros((block, out_width - h), jnp.float32)],
            axis=1).astype(o_ref.dtype)

    return pl.pallas_call(
        body,
        grid=(grid,),
        in_specs=[
            pl.BlockSpec((2, block, h), lambda i: (0, i, 0)),
            pl.BlockSpec((block, d), lambda i: (i, 0)),
            pl.BlockSpec(root.shape, lambda i: (0, 0)),
            pl.BlockSpec((1, h), lambda i: (0, 0)),
        ],
        out_specs=pl.BlockSpec((block, out_width), lambda i: (i, 0)),
        out_shape=jax.ShapeDtypeStruct((n, out_width), jnp.bfloat16),
    )(agg_parts, x, root, b.reshape(1, h))


def _final_head(agg_parts, h1, root2, b2, wd_t, bd):
    n, d = h1.shape
    h = root2.shape[1]

    def body(a_ref, h_ref, r_ref, b_ref, wd_ref, bd_ref, o_ref):
        agg = a_ref[0] + a_ref[1]
        h2 = jnp.maximum(
            agg + jnp.dot(h_ref[...], r_ref[...],
                          preferred_element_type=jnp.float32) + b_ref[...],
            0.0)
        total = jnp.sum(h2 * wd_ref[...]) + bd_ref[0, 0]
        o_ref[...] = jnp.maximum(total, 0.0).reshape(1, 1)

    return pl.pallas_call(
        body,
        grid=(1,),
        in_specs=[
            pl.BlockSpec((2, n, h), lambda i: (0, 0, 0)),
            pl.BlockSpec((n, d), lambda i: (0, 0)),
            pl.BlockSpec(root2.shape, lambda i: (0, 0)),
            pl.BlockSpec((1, h), lambda i: (0, 0)),
            pl.BlockSpec((1, h), lambda i: (0, 0)),
            pl.BlockSpec((1, 1), lambda i: (0, 0)),
        ],
        out_specs=pl.BlockSpec((1, 1), lambda i: (0, 0)),
        out_shape=jax.ShapeDtypeStruct((1, 1), jnp.float32),
    )(agg_parts, h1, root2, b2.reshape(1, h), wd_t, bd.reshape(1, 1))


def kernel(x, edge_index, edge_attr, Wk1, bk1, root1, b1,
           Wk2, bk2, root2, b2, Wd, bd):
    n, f = x.shape
    e, s = edge_attr.shape
    h = root1.shape[1]
    src = edge_index[0]
    dst = edge_index[1]

    ea_ext = jnp.concatenate(
        [edge_attr, jnp.ones((e, 1), jnp.float32)], axis=1)
    w2d1 = jnp.concatenate(
        [jnp.transpose(Wk1, (1, 0, 2)).reshape(f, s * h), bk1], axis=1)
    w2d2 = jnp.concatenate(
        [jnp.transpose(Wk2, (1, 0, 2)).reshape(h, s * h), bk2], axis=1)
    w2d2p = jnp.concatenate(
        [w2d2, jnp.zeros((f - h, w2d2.shape[1]), jnp.float32)], axis=0)
    root2p = jnp.concatenate(
        [root2, jnp.zeros((f - h, h), jnp.float32)], axis=0)
    wd_t = Wd.reshape(1, h)

    xb = x.astype(jnp.bfloat16)
    w2d1b = w2d1.astype(jnp.bfloat16)
    w2d2pb = w2d2p.astype(jnp.bfloat16)
    root2pb = root2p.astype(jnp.bfloat16)

    def to_i32(a):
        return jax.lax.bitcast_convert_type(
            a.reshape(a.shape[0], a.shape[1] // 2, 2), jnp.int32)

    def from_i32(a):
        return jax.lax.bitcast_convert_type(a, jnp.bfloat16).reshape(
            a.shape[0], a.shape[1] * 2)

    hs1 = from_i32(_gather_rows(to_i32(xb), src))
    m1 = _edge_messages(hs1, ea_ext, w2d1b)
    agg1 = _scatter_add_parts(m1, dst, n)
    h1p = _node_update(agg1, x, root1, b1, f)
    hs2 = from_i32(_gather_rows(to_i32(h1p), src))
    m2 = _edge_messages(hs2, ea_ext, w2d2pb)
    agg2 = _scatter_add_parts(m2, dst, n)
    return _final_head(agg2, h1p, root2pb, b2, wd_t, bd)


# speedup vs baseline: 2.1114x; 2.1114x over previous
import functools

import jax
import jax.numpy as jnp
from jax import lax
from jax.experimental import pallas as pl
from jax.experimental.pallas import tpu as pltpu
from jax.experimental.pallas import tpu_sc as plsc

NC = 2
NS = 16
LANES = 16
NW = NC * NS
CH = 128


def _sc_mesh():
    return plsc.VectorSubcoreMesh(
        core_axis_name="c", subcore_axis_name="s",
        num_cores=NC, num_subcores=NS)


NB = 4


def _gather_rows(table, idx):
    n_rows, d = table.shape
    dt = table.dtype
    e = idx.shape[0]
    ew = e // NW
    n_full = ew // CH
    tail = ew - n_full * CH
    n_grp = n_full // NB
    rest = n_full - n_grp * NB

    @functools.partial(
        pl.kernel,
        out_type=jax.ShapeDtypeStruct((e, d), dt),
        mesh=_sc_mesh(),
        scratch_types=[
            pltpu.VMEM((ew,), jnp.int32),
            *[pltpu.VMEM((CH, d), dt) for _ in range(NB)],
            pltpu.VMEM((max(tail, 8), d), dt),
            pltpu.SemaphoreType.DMA,
            pltpu.SemaphoreType.DMA,
        ],
    )
    def k(table_hbm, idx_hbm, out_hbm, idx_all, *bufs):
        *rows, tail_v, gsem, wsem = bufs
        wid = lax.axis_index("s") * NC + lax.axis_index("c")
        base = wid * ew
        pltpu.sync_copy(idx_hbm.at[pl.ds(base, ew)], idx_all)

        def run_group(cb, nb):
            gds = [
                pltpu.async_copy(
                    table_hbm.at[idx_all.at[pl.ds(cb + b * CH, CH)]],
                    rows[b], gsem)
                for b in range(nb)
            ]
            wds = []
            for b in range(nb):
                gds[b].wait()
                wds.append(pltpu.async_copy(
                    rows[b], out_hbm.at[pl.ds(base + cb + b * CH, CH)],
                    wsem))
            for wd in wds:
                wd.wait()

        def body(g, carry):
            run_group(g * (NB * CH), NB)
            return carry

        lax.fori_loop(0, n_grp, body, 0)
        if rest:
            run_group(n_grp * (NB * CH), rest)
        if tail:
            off = n_full * CH
            pltpu.async_copy(
                table_hbm.at[idx_all.at[pl.ds(off, tail)]],
                tail_v.at[pl.ds(0, tail)], gsem).wait()
            pltpu.async_copy(
                tail_v.at[pl.ds(0, tail)],
                out_hbm.at[pl.ds(base + off, tail)], wsem).wait()

    return k(table, idx)


def _scatter_add_parts(vals, dst, n_nodes):
    e, h = vals.shape
    n_chunks = e // CH
    per_w = -(-n_chunks // NW)
    rpt = -(-n_nodes // (8 * NS)) * 8
    npad = rpt * NS

    ew = e // NW
    n_full = ew // CH
    tail = ew - n_full * CH
    n_grp = n_full // NB
    rest = n_full - n_grp * NB

    @functools.partial(
        pl.kernel,
        out_type=jax.ShapeDtypeStruct((NC * npad, h), jnp.float32),
        mesh=_sc_mesh(),
        scratch_types=[
            *[pltpu.VMEM((CH,), jnp.int32) for _ in range(NB)],
            *[pltpu.VMEM((CH, h), jnp.float32) for _ in range(NB)],
            pltpu.VMEM((max(tail, 8),), jnp.int32),
            pltpu.VMEM((max(tail, 8), h), jnp.float32),
            pltpu.VMEM((rpt, h), jnp.float32),
            pltpu.VMEM_SHARED((npad, h), jnp.float32),
            pltpu.SemaphoreType.DMA,
            pltpu.SemaphoreType.DMA,
        ],
        compiler_params=pltpu.CompilerParams(use_tc_tiling_on_sc=False),
    )
    def k(vals_hbm, dst_hbm, out_hbm, *bufs):
        idxs = bufs[:NB]
        rows = bufs[NB:2 * NB]
        tidx, trow, acc_v, acc_sh, lsem, ssem = bufs[2 * NB:]
        cid = lax.axis_index("c")
        sid = lax.axis_index("s")
        wid = sid * NC + cid
        base = wid * ew

        zv = jnp.zeros((LANES,), jnp.float32)

        def zbody(r, carry):
            for j in range(h // LANES):
                acc_v[r, pl.ds(j * LANES, LANES)] = zv
            return carry

        lax.fori_loop(0, rpt, zbody, 0)
        pltpu.sync_copy(acc_v, acc_sh.at[pl.ds(sid * rpt, rpt)])
        plsc.subcore_barrier()

        def run_group(cb, nb):
            lds = []
            for b in range(nb):
                off = base + cb + b * CH
                lds.append((
                    pltpu.async_copy(dst_hbm.at[pl.ds(off, CH)],
                                     idxs[b], lsem),
                    pltpu.async_copy(vals_hbm.at[pl.ds(off, CH)],
                                     rows[b], lsem)))
            sds = []
            for b in range(nb):
                lds[b][0].wait()
                lds[b][1].wait()
                sds.append(pltpu.async_copy(
                    rows[b], acc_sh.at[idxs[b]], ssem, add=True))
            for sd in sds:
                sd.wait()

        def body(g, carry):
            run_group(g * (NB * CH), NB)
            return carry

        lax.fori_loop(0, n_grp, body, 0)
        if rest:
            run_group(n_grp * (NB * CH), rest)
        if tail:
            off = base + n_full * CH
            pltpu.async_copy(dst_hbm.at[pl.ds(off, tail)], tidx, lsem).wait()
            pltpu.async_copy(vals_hbm.at[pl.ds(off, tail)], trow, lsem).wait()
            pltpu.async_copy(trow, acc_sh.at[tidx], ssem, add=True).wait()
        plsc.subcore_barrier()

        pltpu.sync_copy(acc_sh.at[pl.ds(sid * rpt, rpt)], acc_v)
        pltpu.sync_copy(
            acc_v, out_hbm.at[pl.ds(cid * npad + sid * rpt, rpt)])

    return k(vals, dst).reshape(NC, npad, h)[:, :n_nodes, :]


def _edge_messages(hs, ea_ext, w2d, block=2000):
    e, d = hs.shape
    s_ext = ea_ext.shape[1]
    h = w2d.shape[1] // s_ext
    grid = e // block
    r_bcast = jnp.kron(jnp.eye(s_ext, dtype=jnp.float32),
                       jnp.ones((1, h), jnp.float32))
    r_sum = jnp.kron(jnp.ones((s_ext, 1), jnp.float32),
                     jnp.eye(h, dtype=jnp.float32))

    def body(hs_ref, ea_ref, w_ref, rb_ref, rs_ref, o_ref):
        t = jnp.dot(hs_ref[...], w_ref[...],
                    preferred_element_type=jnp.float32)
        eab = jnp.dot(ea_ref[...], rb_ref[...],
                      preferred_element_type=jnp.float32)
        o_ref[...] = jnp.dot(t * eab, rs_ref[...],
                             preferred_element_type=jnp.float32)

    return pl.pallas_call(
        body,
        grid=(grid,),
        in_specs=[
            pl.BlockSpec((block, d), lambda i: (i, 0)),
            pl.BlockSpec((block, s_ext), lambda i: (i, 0)),
            pl.BlockSpec(w2d.shape, lambda i: (0, 0)),
            pl.BlockSpec(r_bcast.shape, lambda i: (0, 0)),
            pl.BlockSpec(r_sum.shape, lambda i: (0, 0)),
        ],
        out_specs=pl.BlockSpec((block, h), lambda i: (i, 0)),
        out_shape=jax.ShapeDtypeStruct((e, h), jnp.float32),
    )(hs, ea_ext, w2d, r_bcast, r_sum)


def _node_update(agg_parts, x, root, b, out_width, block=2000):
    n, d = x.shape
    h = root.shape[1]
    grid = n // block

    def body(a_ref, x_ref, r_ref, b_ref, o_ref):
        agg = a_ref[0] + a_ref[1]
        xr = jnp.dot(x_ref[...], r_ref[...],
                     preferred_element_type=jnp.float32)
        hv = jnp.maximum(agg + xr + b_ref[...], 0.0)
        o_ref[...] = jnp.concatenate(
            [hv, jnp.zeros((block, out_width - h), jnp.float32)],
            axis=1).astype(o_ref.dtype)

    return pl.pallas_call(
        body,
        grid=(grid,),
        in_specs=[
            pl.BlockSpec((2, block, h), lambda i: (0, i, 0)),
            pl.BlockSpec((block, d), lambda i: (i, 0)),
            pl.BlockSpec(root.shape, lambda i: (0, 0)),
            pl.BlockSpec((1, h), lambda i: (0, 0)),
        ],
        out_specs=pl.BlockSpec((block, out_width), lambda i: (i, 0)),
        out_shape=jax.ShapeDtypeStruct((n, out_width), jnp.float32),
    )(agg_parts, x, root, b.reshape(1, h))


def _final_head(agg_parts, h1, root2, b2, wd_t, bd):
    n, d = h1.shape
    h = root2.shape[1]

    def body(a_ref, h_ref, r_ref, b_ref, wd_ref, bd_ref, o_ref):
        agg = a_ref[0] + a_ref[1]
        h2 = jnp.maximum(
            agg + jnp.dot(h_ref[...], r_ref[...],
                          preferred_element_type=jnp.float32) + b_ref[...],
            0.0)
        total = jnp.sum(h2 * wd_ref[...]) + bd_ref[0, 0]
        o_ref[...] = jnp.maximum(total, 0.0).reshape(1, 1)

    return pl.pallas_call(
        body,
        grid=(1,),
        in_specs=[
            pl.BlockSpec((2, n, h), lambda i: (0, 0, 0)),
            pl.BlockSpec((n, d), lambda i: (0, 0)),
            pl.BlockSpec(root2.shape, lambda i: (0, 0)),
            pl.BlockSpec((1, h), lambda i: (0, 0)),
            pl.BlockSpec((1, h), lambda i: (0, 0)),
            pl.BlockSpec((1, 1), lambda i: (0, 0)),
        ],
        out_specs=pl.BlockSpec((1, 1), lambda i: (0, 0)),
        out_shape=jax.ShapeDtypeStruct((1, 1), jnp.float32),
    )(agg_parts, h1, root2, b2.reshape(1, h), wd_t, bd.reshape(1, 1))


def kernel(x, edge_index, edge_attr, Wk1, bk1, root1, b1,
           Wk2, bk2, root2, b2, Wd, bd):
    n, f = x.shape
    e, s = edge_attr.shape
    h = root1.shape[1]
    src = edge_index[0]
    dst = edge_index[1]

    ea_ext = jnp.concatenate(
        [edge_attr, jnp.ones((e, 1), jnp.float32)], axis=1)
    w2d1 = jnp.concatenate(
        [jnp.transpose(Wk1, (1, 0, 2)).reshape(f, s * h), bk1], axis=1)
    w2d2 = jnp.concatenate(
        [jnp.transpose(Wk2, (1, 0, 2)).reshape(h, s * h), bk2], axis=1)
    w2d2p = jnp.concatenate(
        [w2d2, jnp.zeros((f - h, w2d2.shape[1]), jnp.float32)], axis=0)
    root2p = jnp.concatenate(
        [root2, jnp.zeros((f - h, h), jnp.float32)], axis=0)
    wd_t = Wd.reshape(1, h)

    hs1 = _gather_rows(x, src)
    m1 = _edge_messages(hs1, ea_ext, w2d1)
    agg1 = _scatter_add_parts(m1, dst, n)
    h1p = _node_update(agg1, x, root1, b1, f)
    hs2 = _gather_rows(h1p, src)
    m2 = _edge_messages(hs2, ea_ext, w2d2p)
    agg2 = _scatter_add_parts(m2, dst, n)
    return _final_head(agg2, h1p, root2p, b2, wd_t, bd)


# speedup vs baseline: 2.1914x; 1.0379x over previous
import functools

import jax
import jax.numpy as jnp
from jax import lax
from jax.experimental import pallas as pl
from jax.experimental.pallas import tpu as pltpu
from jax.experimental.pallas import tpu_sc as plsc

NC = 2
NS = 16
LANES = 16
NW = NC * NS
CH = 128


def _sc_mesh():
    return plsc.VectorSubcoreMesh(
        core_axis_name="c", subcore_axis_name="s",
        num_cores=NC, num_subcores=NS)


NB = 4


def _gather_rows(table, idx):
    n_rows, d = table.shape
    dt = table.dtype
    e = idx.shape[0]
    ew = e // NW
    n_full = ew // CH
    tail = ew - n_full * CH
    n_grp = n_full // NB
    rest = n_full - n_grp * NB

    @functools.partial(
        pl.kernel,
        out_type=jax.ShapeDtypeStruct((e, d), dt),
        mesh=_sc_mesh(),
        scratch_types=[
            pltpu.VMEM((ew,), jnp.int32),
            *[pltpu.VMEM((CH, d), dt) for _ in range(NB)],
            pltpu.VMEM((max(tail, 8), d), dt),
            pltpu.SemaphoreType.DMA,
            pltpu.SemaphoreType.DMA,
        ],
    )
    def k(table_hbm, idx_hbm, out_hbm, idx_all, *bufs):
        *rows, tail_v, gsem, wsem = bufs
        wid = lax.axis_index("s") * NC + lax.axis_index("c")
        base = wid * ew
        pltpu.sync_copy(idx_hbm.at[pl.ds(base, ew)], idx_all)

        def run_group(cb, nb):
            gds = [
                pltpu.async_copy(
                    table_hbm.at[idx_all.at[pl.ds(cb + b * CH, CH)]],
                    rows[b], gsem)
                for b in range(nb)
            ]
            wds = []
            for b in range(nb):
                gds[b].wait()
                wds.append(pltpu.async_copy(
                    rows[b], out_hbm.at[pl.ds(base + cb + b * CH, CH)],
                    wsem))
            for wd in wds:
                wd.wait()

        def body(g, carry):
            run_group(g * (NB * CH), NB)
            return carry

        lax.fori_loop(0, n_grp, body, 0)
        if rest:
            run_group(n_grp * (NB * CH), rest)
        if tail:
            off = n_full * CH
            pltpu.async_copy(
                table_hbm.at[idx_all.at[pl.ds(off, tail)]],
                tail_v.at[pl.ds(0, tail)], gsem).wait()
            pltpu.async_copy(
                tail_v.at[pl.ds(0, tail)],
                out_hbm.at[pl.ds(base + off, tail)], wsem).wait()

    return k(table, idx)


def _scatter_add_parts(vals, dst, n_nodes):
    e, h = vals.shape
    n_chunks = e // CH
    per_w = -(-n_chunks // NW)
    rpt = -(-n_nodes // (8 * NS)) * 8
    npad = rpt * NS

    ew = e // NW
    n_full = ew // CH
    tail = ew - n_full * CH
    n_grp = n_full // NB
    rest = n_full - n_grp * NB

    @functools.partial(
        pl.kernel,
        out_type=jax.ShapeDtypeStruct((NC * npad, h), jnp.float32),
        mesh=_sc_mesh(),
        scratch_types=[
            *[pltpu.VMEM((CH,), jnp.int32) for _ in range(NB)],
            *[pltpu.VMEM((CH, h), jnp.float32) for _ in range(NB)],
            pltpu.VMEM((max(tail, 8),), jnp.int32),
            pltpu.VMEM((max(tail, 8), h), jnp.float32),
            pltpu.VMEM((rpt, h), jnp.float32),
            pltpu.VMEM_SHARED((npad, h), jnp.float32),
            pltpu.SemaphoreType.DMA,
            pltpu.SemaphoreType.DMA,
        ],
        compiler_params=pltpu.CompilerParams(use_tc_tiling_on_sc=False),
    )
    def k(vals_hbm, dst_hbm, out_hbm, *bufs):
        idxs = bufs[:NB]
        rows = bufs[NB:2 * NB]
        tidx, trow, acc_v, acc_sh, lsem, ssem = bufs[2 * NB:]
        cid = lax.axis_index("c")
        sid = lax.axis_index("s")
        wid = sid * NC + cid
        base = wid * ew

        zv = jnp.zeros((LANES,), jnp.float32)

        def zbody(r, carry):
            for j in range(h // LANES):
                acc_v[r, pl.ds(j * LANES, LANES)] = zv
            return carry

        lax.fori_loop(0, rpt, zbody, 0)
        pltpu.sync_copy(acc_v, acc_sh.at[pl.ds(sid * rpt, rpt)])
        plsc.subcore_barrier()

        def run_group(cb, nb):
            lds = []
            for b in range(nb):
                off = base + cb + b * CH
                lds.append((
                    pltpu.async_copy(dst_hbm.at[pl.ds(off, CH)],
                                     idxs[b], lsem),
                    pltpu.async_copy(vals_hbm.at[pl.ds(off, CH)],
                                     rows[b], lsem)))
            sds = []
            for b in range(nb):
                lds[b][0].wait()
                lds[b][1].wait()
                sds.append(pltpu.async_copy(
                    rows[b], acc_sh.at[idxs[b]], ssem, add=True))
            for sd in sds:
                sd.wait()

        def body(g, carry):
            run_group(g * (NB * CH), NB)
            return carry

        lax.fori_loop(0, n_grp, body, 0)
        if rest:
            run_group(n_grp * (NB * CH), rest)
        if tail:
            off = base + n_full * CH
            pltpu.async_copy(dst_hbm.at[pl.ds(off, tail)], tidx, lsem).wait()
            pltpu.async_copy(vals_hbm.at[pl.ds(off, tail)], trow, lsem).wait()
            pltpu.async_copy(trow, acc_sh.at[tidx], ssem, add=True).wait()
        plsc.subcore_barrier()

        pltpu.sync_copy(acc_sh.at[pl.ds(sid * rpt, rpt)], acc_v)
        pltpu.sync_copy(
            acc_v, out_hbm.at[pl.ds(cid * npad + sid * rpt, rpt)])

    return k(vals, dst).reshape(NC, npad, h)[:, :n_nodes, :]


def _edge_messages(hs, ea_ext, w2d, block=2000):
    e, d = hs.shape
    s_ext = ea_ext.shape[1] + 1
    h = w2d.shape[1] // s_ext
    grid = e // block
    r_full = jnp.kron(jnp.eye(s_ext, dtype=jnp.float32),
                      jnp.ones((1, h), jnp.float32))
    r_bcast = r_full[:s_ext - 1]
    r_bias = r_full[s_ext - 1:]
    r_sum = jnp.kron(jnp.ones((s_ext, 1), jnp.float32),
                     jnp.eye(h, dtype=jnp.float32))

    def body(hs_ref, ea_ref, w_ref, rb_ref, rbias_ref, rs_ref, o_ref):
        t = jnp.dot(hs_ref[...], w_ref[...],
                    preferred_element_type=jnp.float32)
        eab = jnp.dot(ea_ref[...], rb_ref[...],
                      preferred_element_type=jnp.float32) + rbias_ref[...]
        o_ref[...] = jnp.dot(t * eab, rs_ref[...],
                             preferred_element_type=jnp.float32)

    return pl.pallas_call(
        body,
        grid=(grid,),
        in_specs=[
            pl.BlockSpec((block, d), lambda i: (i, 0)),
            pl.BlockSpec((block, s_ext - 1), lambda i: (i, 0)),
            pl.BlockSpec(w2d.shape, lambda i: (0, 0)),
            pl.BlockSpec(r_bcast.shape, lambda i: (0, 0)),
            pl.BlockSpec(r_bias.shape, lambda i: (0, 0)),
            pl.BlockSpec(r_sum.shape, lambda i: (0, 0)),
        ],
        out_specs=pl.BlockSpec((block, h), lambda i: (i, 0)),
        out_shape=jax.ShapeDtypeStruct((e, h), jnp.float32),
    )(hs, ea_ext, w2d, r_bcast, r_bias, r_sum)


def _node_update(agg_parts, x, root, b, out_width, block=2000):
    n, d = x.shape
    h = root.shape[1]
    grid = n // block

    def body(a_ref, x_ref, r_ref, b_ref, o_ref):
        agg = a_ref[0] + a_ref[1]
        xr = jnp.dot(x_ref[...], r_ref[...],
                     preferred_element_type=jnp.float32)
        hv = jnp.maximum(agg + xr + b_ref[...], 0.0)
        o_ref[...] = jnp.concatenate(
            [hv, jnp.zeros((block, out_width - h), jnp.float32)],
            axis=1).astype(o_ref.dtype)

    return pl.pallas_call(
        body,
        grid=(grid,),
        in_specs=[
            pl.BlockSpec((2, block, h), lambda i: (0, i, 0)),
            pl.BlockSpec((block, d), lambda i: (i, 0)),
            pl.BlockSpec(root.shape, lambda i: (0, 0)),
            pl.BlockSpec((1, h), lambda i: (0, 0)),
        ],
        out_specs=pl.BlockSpec((block, out_width), lambda i: (i, 0)),
        out_shape=jax.ShapeDtypeStruct((n, out_width), jnp.float32),
    )(agg_parts, x, root, b.reshape(1, h))


def _final_head(agg_parts, h1, root2, b2, wd_t, bd):
    n, d = h1.shape
    h = root2.shape[1]

    def body(a_ref, h_ref, r_ref, b_ref, wd_ref, bd_ref, o_ref):
        agg = a_ref[0] + a_ref[1]
        h2 = jnp.maximum(
            agg + jnp.dot(h_ref[...], r_ref[...],
                          preferred_element_type=jnp.float32) + b_ref[...],
            0.0)
        total = jnp.sum(h2 * wd_ref[...]) + bd_ref[0, 0]
        o_ref[...] = jnp.maximum(total, 0.0).reshape(1, 1)

    return pl.pallas_call(
        body,
        grid=(1,),
        in_specs=[
            pl.BlockSpec((2, n, h), lambda i: (0, 0, 0)),
            pl.BlockSpec((n, d), lambda i: (0, 0)),
            pl.BlockSpec(root2.shape, lambda i: (0, 0)),
            pl.BlockSpec((1, h), lambda i: (0, 0)),
            pl.BlockSpec((1, h), lambda i: (0, 0)),
            pl.BlockSpec((1, 1), lambda i: (0, 0)),
        ],
        out_specs=pl.BlockSpec((1, 1), lambda i: (0, 0)),
        out_shape=jax.ShapeDtypeStruct((1, 1), jnp.float32),
    )(agg_parts, h1, root2, b2.reshape(1, h), wd_t, bd.reshape(1, 1))


def kernel(x, edge_index, edge_attr, Wk1, bk1, root1, b1,
           Wk2, bk2, root2, b2, Wd, bd):
    n, f = x.shape
    e, s = edge_attr.shape
    h = root1.shape[1]
    src = edge_index[0]
    dst = edge_index[1]

    ea_ext = edge_attr
    w2d1 = jnp.concatenate(
        [jnp.transpose(Wk1, (1, 0, 2)).reshape(f, s * h), bk1], axis=1)
    w2d2 = jnp.concatenate(
        [jnp.transpose(Wk2, (1, 0, 2)).reshape(h, s * h), bk2], axis=1)
    w2d2p = jnp.concatenate(
        [w2d2, jnp.zeros((f - h, w2d2.shape[1]), jnp.float32)], axis=0)
    root2p = jnp.concatenate(
        [root2, jnp.zeros((f - h, h), jnp.float32)], axis=0)
    wd_t = Wd.reshape(1, h)

    hs1 = _gather_rows(x, src)
    m1 = _edge_messages(hs1, ea_ext, w2d1)
    agg1 = _scatter_add_parts(m1, dst, n)
    h1p = _node_update(agg1, x, root1, b1, f)
    hs2 = _gather_rows(h1p, src)
    m2 = _edge_messages(hs2, ea_ext, w2d2p)
    agg2 = _scatter_add_parts(m2, dst, n)
    return _final_head(agg2, h1p, root2p, b2, wd_t, bd)


# speedup vs baseline: 2.2324x; 1.0187x over previous
import functools

import jax
import jax.numpy as jnp
from jax import lax
from jax.experimental import pallas as pl
from jax.experimental.pallas import tpu as pltpu
from jax.experimental.pallas import tpu_sc as plsc

NC = 2
NS = 16
LANES = 16
NW = NC * NS
CH = 128


def _sc_mesh():
    return plsc.VectorSubcoreMesh(
        core_axis_name="c", subcore_axis_name="s",
        num_cores=NC, num_subcores=NS)


NB = 4


def _gather_rows(table, idx):
    n_rows, d = table.shape
    dt = table.dtype
    e = idx.shape[0]
    ew = e // NW
    n_full = ew // CH
    tail = ew - n_full * CH
    n_grp = n_full // NB
    rest = n_full - n_grp * NB

    @functools.partial(
        pl.kernel,
        out_type=jax.ShapeDtypeStruct((e, d), dt),
        mesh=_sc_mesh(),
        scratch_types=[
            pltpu.VMEM((ew,), jnp.int32),
            *[pltpu.VMEM((CH, d), dt) for _ in range(NB)],
            pltpu.VMEM((max(tail, 8), d), dt),
            pltpu.SemaphoreType.DMA,
            pltpu.SemaphoreType.DMA,
        ],
    )
    def k(table_hbm, idx_hbm, out_hbm, idx_all, *bufs):
        *rows, tail_v, gsem, wsem = bufs
        wid = lax.axis_index("s") * NC + lax.axis_index("c")
        base = wid * ew
        pltpu.sync_copy(idx_hbm.at[pl.ds(base, ew)], idx_all)

        def run_group(cb, nb):
            gds = [
                pltpu.async_copy(
                    table_hbm.at[idx_all.at[pl.ds(cb + b * CH, CH)]],
                    rows[b], gsem)
                for b in range(nb)
            ]
            wds = []
            for b in range(nb):
                gds[b].wait()
                wds.append(pltpu.async_copy(
                    rows[b], out_hbm.at[pl.ds(base + cb + b * CH, CH)],
                    wsem))
            for wd in wds:
                wd.wait()

        def body(g, carry):
            run_group(g * (NB * CH), NB)
            return carry

        lax.fori_loop(0, n_grp, body, 0)
        if rest:
            run_group(n_grp * (NB * CH), rest)
        if tail:
            off = n_full * CH
            pltpu.async_copy(
                table_hbm.at[idx_all.at[pl.ds(off, tail)]],
                tail_v.at[pl.ds(0, tail)], gsem).wait()
            pltpu.async_copy(
                tail_v.at[pl.ds(0, tail)],
                out_hbm.at[pl.ds(base + off, tail)], wsem).wait()

    return k(table, idx)


def _scatter_add_parts(vals, dst, n_nodes):
    e, h = vals.shape
    n_chunks = e // CH
    per_w = -(-n_chunks // NW)
    rpt = -(-n_nodes // (8 * NS)) * 8
    npad = rpt * NS

    ew = e // NW
    n_full = ew // CH
    tail = ew - n_full * CH
    n_grp = n_full // NB
    rest = n_full - n_grp * NB

    @functools.partial(
        pl.kernel,
        out_type=jax.ShapeDtypeStruct((NC * npad, h), jnp.float32),
        mesh=_sc_mesh(),
        scratch_types=[
            *[pltpu.VMEM((CH,), jnp.int32) for _ in range(NB)],
            *[pltpu.VMEM((CH, h), jnp.float32) for _ in range(NB)],
            pltpu.VMEM((max(tail, 8),), jnp.int32),
            pltpu.VMEM((max(tail, 8), h), jnp.float32),
            pltpu.VMEM((rpt, h), jnp.float32),
            pltpu.VMEM_SHARED((npad, h), jnp.float32),
            pltpu.SemaphoreType.DMA,
            pltpu.SemaphoreType.DMA,
        ],
        compiler_params=pltpu.CompilerParams(use_tc_tiling_on_sc=False),
    )
    def k(vals_hbm, dst_hbm, out_hbm, *bufs):
        idxs = bufs[:NB]
        rows = bufs[NB:2 * NB]
        tidx, trow, acc_v, acc_sh, lsem, ssem = bufs[2 * NB:]
        cid = lax.axis_index("c")
        sid = lax.axis_index("s")
        wid = sid * NC + cid
        base = wid * ew

        zv = jnp.zeros((LANES,), jnp.float32)

        def zbody(r, carry):
            for j in range(h // LANES):
                acc_v[r, pl.ds(j * LANES, LANES)] = zv
            return carry

        lax.fori_loop(0, rpt, zbody, 0)
        pltpu.sync_copy(acc_v, acc_sh.at[pl.ds(sid * rpt, rpt)])
        plsc.subcore_barrier()

        def run_group(cb, nb):
            lds = []
            for b in range(nb):
                off = base + cb + b * CH
                lds.append((
                    pltpu.async_copy(dst_hbm.at[pl.ds(off, CH)],
                                     idxs[b], lsem),
                    pltpu.async_copy(vals_hbm.at[pl.ds(off, CH)],
                                     rows[b], lsem)))
            sds = []
            for b in range(nb):
                lds[b][0].wait()
                lds[b][1].wait()
                sds.append(pltpu.async_copy(
                    rows[b], acc_sh.at[idxs[b]], ssem, add=True))
            for sd in sds:
                sd.wait()

        def body(g, carry):
            run_group(g * (NB * CH), NB)
            return carry

        lax.fori_loop(0, n_grp, body, 0)
        if rest:
            run_group(n_grp * (NB * CH), rest)
        if tail:
            off = base + n_full * CH
            pltpu.async_copy(dst_hbm.at[pl.ds(off, tail)], tidx, lsem).wait()
            pltpu.async_copy(vals_hbm.at[pl.ds(off, tail)], trow, lsem).wait()
            pltpu.async_copy(trow, acc_sh.at[tidx], ssem, add=True).wait()
        plsc.subcore_barrier()

        pltpu.sync_copy(acc_sh.at[pl.ds(sid * rpt, rpt)], acc_v)
        pltpu.sync_copy(
            acc_v, out_hbm.at[pl.ds(cid * npad + sid * rpt, rpt)])

    return k(vals, dst).reshape(NC, npad, h)[:, :n_nodes, :]


def _edge_messages(hs, ea_ext, w2d, block=2000):
    e, d = hs.shape
    s_ext = ea_ext.shape[1] + 1
    h = w2d.shape[1] // s_ext
    grid = e // block
    r_full = jnp.kron(jnp.eye(s_ext, dtype=jnp.float32),
                      jnp.ones((1, h), jnp.float32))
    r_bcast = r_full[:s_ext - 1]
    r_bias = r_full[s_ext - 1:]
    r_sum = jnp.kron(jnp.ones((s_ext, 1), jnp.float32),
                     jnp.eye(h, dtype=jnp.float32))

    def body(hs_ref, ea_ref, w_ref, rb_ref, rbias_ref, rs_ref, o_ref):
        t = jnp.dot(hs_ref[...], w_ref[...],
                    preferred_element_type=jnp.float32)
        eab = jnp.dot(ea_ref[...], rb_ref[...],
                      preferred_element_type=jnp.float32) + rbias_ref[...]
        o_ref[...] = jnp.dot(t * eab, rs_ref[...],
                             preferred_element_type=jnp.float32)

    return pl.pallas_call(
        body,
        grid=(grid,),
        in_specs=[
            pl.BlockSpec((block, d), lambda i: (i, 0)),
            pl.BlockSpec((block, s_ext - 1), lambda i: (i, 0)),
            pl.BlockSpec(w2d.shape, lambda i: (0, 0)),
            pl.BlockSpec(r_bcast.shape, lambda i: (0, 0)),
            pl.BlockSpec(r_bias.shape, lambda i: (0, 0)),
            pl.BlockSpec(r_sum.shape, lambda i: (0, 0)),
        ],
        out_specs=pl.BlockSpec((block, h), lambda i: (i, 0)),
        out_shape=jax.ShapeDtypeStruct((e, h), jnp.float32),
    )(hs, ea_ext, w2d, r_bcast, r_bias, r_sum)


def _node_update(agg_parts, x, root, b, out_width, block=2000):
    n, d = x.shape
    h = root.shape[1]
    grid = n // block

    n_parts = len(agg_parts)

    def body(*refs):
        p_refs = refs[:n_parts]
        x_ref, r_ref, b_ref, o_ref = refs[n_parts:]
        agg = p_refs[0][0] + p_refs[0][1]
        for pr in p_refs[1:]:
            agg = agg + pr[0] + pr[1]
        xr = jnp.dot(x_ref[...], r_ref[...],
                     preferred_element_type=jnp.float32)
        hv = jnp.maximum(agg + xr + b_ref[...], 0.0)
        o_ref[...] = jnp.concatenate(
            [hv, jnp.zeros((block, out_width - h), jnp.float32)],
            axis=1).astype(o_ref.dtype)

    return pl.pallas_call(
        body,
        grid=(grid,),
        in_specs=[
            *[pl.BlockSpec((2, block, h), lambda i: (0, i, 0))
              for _ in range(n_parts)],
            pl.BlockSpec((block, d), lambda i: (i, 0)),
            pl.BlockSpec(root.shape, lambda i: (0, 0)),
            pl.BlockSpec((1, h), lambda i: (0, 0)),
        ],
        out_specs=pl.BlockSpec((block, out_width), lambda i: (i, 0)),
        out_shape=jax.ShapeDtypeStruct((n, out_width), jnp.float32),
    )(*agg_parts, x, root, b.reshape(1, h))


def _final_head(agg_parts, h1, root2, b2, wd_t, bd):
    n, d = h1.shape
    h = root2.shape[1]
    n_parts = len(agg_parts)

    def body(*refs):
        p_refs = refs[:n_parts]
        h_ref, r_ref, b_ref, wd_ref, bd_ref, o_ref = refs[n_parts:]
        agg = p_refs[0][0] + p_refs[0][1]
        for pr in p_refs[1:]:
            agg = agg + pr[0] + pr[1]
        h2 = jnp.maximum(
            agg + jnp.dot(h_ref[...], r_ref[...],
                          preferred_element_type=jnp.float32) + b_ref[...],
            0.0)
        total = jnp.sum(h2 * wd_ref[...]) + bd_ref[0, 0]
        o_ref[...] = jnp.maximum(total, 0.0).reshape(1, 1)

    return pl.pallas_call(
        body,
        grid=(1,),
        in_specs=[
            *[pl.BlockSpec((2, n, h), lambda i: (0, 0, 0))
              for _ in range(n_parts)],
            pl.BlockSpec((n, d), lambda i: (0, 0)),
            pl.BlockSpec(root2.shape, lambda i: (0, 0)),
            pl.BlockSpec((1, h), lambda i: (0, 0)),
            pl.BlockSpec((1, h), lambda i: (0, 0)),
            pl.BlockSpec((1, 1), lambda i: (0, 0)),
        ],
        out_specs=pl.BlockSpec((1, 1), lambda i: (0, 0)),
        out_shape=jax.ShapeDtypeStruct((1, 1), jnp.float32),
    )(*agg_parts, h1, root2, b2.reshape(1, h), wd_t, bd.reshape(1, 1))


def kernel(x, edge_index, edge_attr, Wk1, bk1, root1, b1,
           Wk2, bk2, root2, b2, Wd, bd):
    n, f = x.shape
    e, s = edge_attr.shape
    h = root1.shape[1]
    src = edge_index[0]
    dst = edge_index[1]

    ea_ext = edge_attr
    w2d1 = jnp.concatenate(
        [jnp.transpose(Wk1, (1, 0, 2)).reshape(f, s * h), bk1], axis=1)
    w2d2 = jnp.concatenate(
        [jnp.transpose(Wk2, (1, 0, 2)).reshape(h, s * h), bk2], axis=1)
    w2d2p = jnp.concatenate(
        [w2d2, jnp.zeros((f - h, w2d2.shape[1]), jnp.float32)], axis=0)
    root2p = jnp.concatenate(
        [root2, jnp.zeros((f - h, h), jnp.float32)], axis=0)
    wd_t = Wd.reshape(1, h)

    e0 = (e // 2) // 1280 * 1280
    halves = [(0, e0), (e0, e - e0)]

    def layer(table, w2d_l):
        parts = []
        for off, cnt in halves:
            hs = _gather_rows(table, src[off:off + cnt])
            m = _edge_messages(hs, edge_attr[off:off + cnt], w2d_l,
                               block=cnt // 40)
            parts.append(_scatter_add_parts(m, dst[off:off + cnt], n))
        return parts

    agg1 = layer(x, w2d1)
    h1p = _node_update(agg1, x, root1, b1, f)
    agg2 = layer(h1p, w2d2p)
    return _final_head(agg2, h1p, root2p, b2, wd_t, bd)


# speedup vs baseline: 2.2435x; 1.0050x over previous
import functools

import jax
import jax.numpy as jnp
from jax import lax
from jax.experimental import pallas as pl
from jax.experimental.pallas import tpu as pltpu
from jax.experimental.pallas import tpu_sc as plsc

NC = 2
NS = 16
LANES = 16
NW = NC * NS
CH = 128


def _sc_mesh():
    return plsc.VectorSubcoreMesh(
        core_axis_name="c", subcore_axis_name="s",
        num_cores=NC, num_subcores=NS)


NB = 6


def _gather_rows(table, idx):
    n_rows, d = table.shape
    dt = table.dtype
    e = idx.shape[0]
    ew = e // NW
    n_full = ew // CH
    tail = ew - n_full * CH
    n_grp = n_full // NB
    rest = n_full - n_grp * NB

    @functools.partial(
        pl.kernel,
        out_type=jax.ShapeDtypeStruct((e, d), dt),
        mesh=_sc_mesh(),
        scratch_types=[
            pltpu.VMEM((ew,), jnp.int32),
            *[pltpu.VMEM((CH, d), dt) for _ in range(NB)],
            pltpu.VMEM((max(tail, 8), d), dt),
            pltpu.SemaphoreType.DMA,
            pltpu.SemaphoreType.DMA,
        ],
    )
    def k(table_hbm, idx_hbm, out_hbm, idx_all, *bufs):
        *rows, tail_v, gsem, wsem = bufs
        wid = lax.axis_index("s") * NC + lax.axis_index("c")
        base = wid * ew
        pltpu.sync_copy(idx_hbm.at[pl.ds(base, ew)], idx_all)

        def run_group(cb, nb):
            gds = [
                pltpu.async_copy(
                    table_hbm.at[idx_all.at[pl.ds(cb + b * CH, CH)]],
                    rows[b], gsem)
                for b in range(nb)
            ]
            wds = []
            for b in range(nb):
                gds[b].wait()
                wds.append(pltpu.async_copy(
                    rows[b], out_hbm.at[pl.ds(base + cb + b * CH, CH)],
                    wsem))
            for wd in wds:
                wd.wait()

        def body(g, carry):
            run_group(g * (NB * CH), NB)
            return carry

        lax.fori_loop(0, n_grp, body, 0)
        if rest:
            run_group(n_grp * (NB * CH), rest)
        if tail:
            off = n_full * CH
            pltpu.async_copy(
                table_hbm.at[idx_all.at[pl.ds(off, tail)]],
                tail_v.at[pl.ds(0, tail)], gsem).wait()
            pltpu.async_copy(
                tail_v.at[pl.ds(0, tail)],
                out_hbm.at[pl.ds(base + off, tail)], wsem).wait()

    return k(table, idx)


def _scatter_add_parts(vals, dst, n_nodes):
    e, h = vals.shape
    n_chunks = e // CH
    per_w = -(-n_chunks // NW)
    rpt = -(-n_nodes // (8 * NS)) * 8
    npad = rpt * NS

    ew = e // NW
    n_full = ew // CH
    tail = ew - n_full * CH
    n_grp = n_full // NB
    rest = n_full - n_grp * NB

    @functools.partial(
        pl.kernel,
        out_type=jax.ShapeDtypeStruct((NC * npad, h), jnp.float32),
        mesh=_sc_mesh(),
        scratch_types=[
            *[pltpu.VMEM((CH,), jnp.int32) for _ in range(NB)],
            *[pltpu.VMEM((CH, h), jnp.float32) for _ in range(NB)],
            pltpu.VMEM((max(tail, 8),), jnp.int32),
            pltpu.VMEM((max(tail, 8), h), jnp.float32),
            pltpu.VMEM((rpt, h), jnp.float32),
            pltpu.VMEM_SHARED((npad, h), jnp.float32),
            pltpu.SemaphoreType.DMA,
            pltpu.SemaphoreType.DMA,
        ],
        compiler_params=pltpu.CompilerParams(use_tc_tiling_on_sc=False),
    )
    def k(vals_hbm, dst_hbm, out_hbm, *bufs):
        idxs = bufs[:NB]
        rows = bufs[NB:2 * NB]
        tidx, trow, acc_v, acc_sh, lsem, ssem = bufs[2 * NB:]
        cid = lax.axis_index("c")
        sid = lax.axis_index("s")
        wid = sid * NC + cid
        base = wid * ew

        zv = jnp.zeros((LANES,), jnp.float32)

        def zbody(r, carry):
            for j in range(h // LANES):
                acc_v[r, pl.ds(j * LANES, LANES)] = zv
            return carry

        lax.fori_loop(0, rpt, zbody, 0)
        pltpu.sync_copy(acc_v, acc_sh.at[pl.ds(sid * rpt, rpt)])
        plsc.subcore_barrier()

        def run_group(cb, nb):
            lds = []
            for b in range(nb):
                off = base + cb + b * CH
                lds.append((
                    pltpu.async_copy(dst_hbm.at[pl.ds(off, CH)],
                                     idxs[b], lsem),
                    pltpu.async_copy(vals_hbm.at[pl.ds(off, CH)],
                                     rows[b], lsem)))
            sds = []
            for b in range(nb):
                lds[b][0].wait()
                lds[b][1].wait()
                sds.append(pltpu.async_copy(
                    rows[b], acc_sh.at[idxs[b]], ssem, add=True))
            for sd in sds:
                sd.wait()

        def body(g, carry):
            run_group(g * (NB * CH), NB)
            return carry

        lax.fori_loop(0, n_grp, body, 0)
        if rest:
            run_group(n_grp * (NB * CH), rest)
        if tail:
            off = base + n_full * CH
            pltpu.async_copy(dst_hbm.at[pl.ds(off, tail)], tidx, lsem).wait()
            pltpu.async_copy(vals_hbm.at[pl.ds(off, tail)], trow, lsem).wait()
            pltpu.async_copy(trow, acc_sh.at[tidx], ssem, add=True).wait()
        plsc.subcore_barrier()

        pltpu.sync_copy(acc_sh.at[pl.ds(sid * rpt, rpt)], acc_v)
        pltpu.sync_copy(
            acc_v, out_hbm.at[pl.ds(cid * npad + sid * rpt, rpt)])

    return k(vals, dst).reshape(NC, npad, h)[:, :n_nodes, :]


def _edge_messages(hs, ea_ext, w2d, block=2000):
    e, d = hs.shape
    s_ext = ea_ext.shape[1] + 1
    h = w2d.shape[1] // s_ext
    grid = e // block
    r_full = jnp.kron(jnp.eye(s_ext, dtype=jnp.float32),
                      jnp.ones((1, h), jnp.float32))
    r_bcast = r_full[:s_ext - 1]
    r_bias = r_full[s_ext - 1:]
    r_sum = jnp.kron(jnp.ones((s_ext, 1), jnp.float32),
                     jnp.eye(h, dtype=jnp.float32))

    def body(hs_ref, ea_ref, w_ref, rb_ref, rbias_ref, rs_ref, o_ref):
        t = jnp.dot(hs_ref[...], w_ref[...],
                    preferred_element_type=jnp.float32)
        eab = jnp.dot(ea_ref[...], rb_ref[...],
                      preferred_element_type=jnp.float32) + rbias_ref[...]
        o_ref[...] = jnp.dot(t * eab, rs_ref[...],
                             preferred_element_type=jnp.float32)

    return pl.pallas_call(
        body,
        grid=(grid,),
        in_specs=[
            pl.BlockSpec((block, d), lambda i: (i, 0)),
            pl.BlockSpec((block, s_ext - 1), lambda i: (i, 0)),
            pl.BlockSpec(w2d.shape, lambda i: (0, 0)),
            pl.BlockSpec(r_bcast.shape, lambda i: (0, 0)),
            pl.BlockSpec(r_bias.shape, lambda i: (0, 0)),
            pl.BlockSpec(r_sum.shape, lambda i: (0, 0)),
        ],
        out_specs=pl.BlockSpec((block, h), lambda i: (i, 0)),
        out_shape=jax.ShapeDtypeStruct((e, h), jnp.float32),
    )(hs, ea_ext, w2d, r_bcast, r_bias, r_sum)


def _node_update(agg_parts, x, root, b, out_width, block=2000):
    n, d = x.shape
    h = root.shape[1]
    grid = n // block

    n_parts = len(agg_parts)

    def body(*refs):
        p_refs = refs[:n_parts]
        x_ref, r_ref, b_ref, o_ref = refs[n_parts:]
        agg = p_refs[0][0] + p_refs[0][1]
        for pr in p_refs[1:]:
            agg = agg + pr[0] + pr[1]
        xr = jnp.dot(x_ref[...], r_ref[...],
                     preferred_element_type=jnp.float32)
        hv = jnp.maximum(agg + xr + b_ref[...], 0.0)
        o_ref[...] = jnp.concatenate(
            [hv, jnp.zeros((block, out_width - h), jnp.float32)],
            axis=1).astype(o_ref.dtype)

    return pl.pallas_call(
        body,
        grid=(grid,),
        in_specs=[
            *[pl.BlockSpec((2, block, h), lambda i: (0, i, 0))
              for _ in range(n_parts)],
            pl.BlockSpec((block, d), lambda i: (i, 0)),
            pl.BlockSpec(root.shape, lambda i: (0, 0)),
            pl.BlockSpec((1, h), lambda i: (0, 0)),
        ],
        out_specs=pl.BlockSpec((block, out_width), lambda i: (i, 0)),
        out_shape=jax.ShapeDtypeStruct((n, out_width), jnp.float32),
    )(*agg_parts, x, root, b.reshape(1, h))


def _final_head(agg_parts, h1, root2, b2, wd_t, bd):
    n, d = h1.shape
    h = root2.shape[1]
    n_parts = len(agg_parts)

    def body(*refs):
        p_refs = refs[:n_parts]
        h_ref, r_ref, b_ref, wd_ref, bd_ref, o_ref = refs[n_parts:]
        agg = p_refs[0][0] + p_refs[0][1]
        for pr in p_refs[1:]:
            agg = agg + pr[0] + pr[1]
        h2 = jnp.maximum(
            agg + jnp.dot(h_ref[...], r_ref[...],
                          preferred_element_type=jnp.float32) + b_ref[...],
            0.0)
        total = jnp.sum(h2 * wd_ref[...]) + bd_ref[0, 0]
        o_ref[...] = jnp.maximum(total, 0.0).reshape(1, 1)

    return pl.pallas_call(
        body,
        grid=(1,),
        in_specs=[
            *[pl.BlockSpec((2, n, h), lambda i: (0, 0, 0))
              for _ in range(n_parts)],
            pl.BlockSpec((n, d), lambda i: (0, 0)),
            pl.BlockSpec(root2.shape, lambda i: (0, 0)),
            pl.BlockSpec((1, h), lambda i: (0, 0)),
            pl.BlockSpec((1, h), lambda i: (0, 0)),
            pl.BlockSpec((1, 1), lambda i: (0, 0)),
        ],
        out_specs=pl.BlockSpec((1, 1), lambda i: (0, 0)),
        out_shape=jax.ShapeDtypeStruct((1, 1), jnp.float32),
    )(*agg_parts, h1, root2, b2.reshape(1, h), wd_t, bd.reshape(1, 1))


def kernel(x, edge_index, edge_attr, Wk1, bk1, root1, b1,
           Wk2, bk2, root2, b2, Wd, bd):
    n, f = x.shape
    e, s = edge_attr.shape
    h = root1.shape[1]
    src = edge_index[0]
    dst = edge_index[1]

    ea_ext = edge_attr
    w2d1 = jnp.concatenate(
        [jnp.transpose(Wk1, (1, 0, 2)).reshape(f, s * h), bk1], axis=1)
    w2d2 = jnp.concatenate(
        [jnp.transpose(Wk2, (1, 0, 2)).reshape(h, s * h), bk2], axis=1)
    w2d2p = jnp.concatenate(
        [w2d2, jnp.zeros((f - h, w2d2.shape[1]), jnp.float32)], axis=0)
    root2p = jnp.concatenate(
        [root2, jnp.zeros((f - h, h), jnp.float32)], axis=0)
    wd_t = Wd.reshape(1, h)

    e0 = (e // 2) // 1280 * 1280
    halves = [(0, e0), (e0, e - e0)]

    def layer(table, w2d_l):
        parts = []
        for off, cnt in halves:
            hs = _gather_rows(table, src[off:off + cnt])
            m = _edge_messages(hs, edge_attr[off:off + cnt], w2d_l,
                               block=cnt // 40)
            parts.append(_scatter_add_parts(m, dst[off:off + cnt], n))
        return parts

    agg1 = layer(x, w2d1)
    h1p = _node_update(agg1, x, root1, b1, f)
    agg2 = layer(h1p, w2d2p)
    return _final_head(agg2, h1p, root2p, b2, wd_t, bd)
